# Initial kernel scaffold; baseline (speedup 1.0000x reference)
import jax
import jax.numpy as jnp
from jax.experimental import pallas as pl


def kernel(words, table):
    raise NotImplementedError("write your pallas kernel here")


# speedup vs baseline: 2.7220x; 2.7220x over previous
import functools

import jax
import jax.numpy as jnp
from jax import lax
from jax.experimental import pallas as pl
from jax.experimental.pallas import tpu as pltpu
from jax.experimental.pallas import tpu_sc as plsc

EMBED_DIM = 128
CHUNK = 128
NUM_CORES = 2
NUM_SUBCORES = 16
NUM_WORKERS = NUM_CORES * NUM_SUBCORES


@functools.cache
def _make_gather(B: int, D: int):
    assert B % (NUM_WORKERS * CHUNK) == 0
    chunks_per_w = B // (NUM_WORKERS * CHUNK)
    mesh = plsc.VectorSubcoreMesh(core_axis_name="c", subcore_axis_name="s")

    @functools.partial(
        pl.kernel,
        mesh=mesh,
        out_type=jax.ShapeDtypeStruct((B, D), jnp.float32),
        scratch_types=[
            pltpu.VMEM((chunks_per_w, CHUNK), jnp.int32),
            pltpu.VMEM((CHUNK, D), jnp.float32),
            pltpu.SemaphoreType.DMA,
        ],
    )
    def gather_kernel(table_hbm, idx_hbm, out_hbm, idx_v, rows_v, sem):
        wid = lax.axis_index("s") * NUM_CORES + lax.axis_index("c")
        pltpu.sync_copy(idx_hbm.at[wid], idx_v)

        def body(j, carry):
            pltpu.async_copy(table_hbm.at[idx_v.at[j]], rows_v, sem).wait()
            out_base = pl.multiple_of((wid * chunks_per_w + j) * CHUNK, CHUNK)
            pltpu.sync_copy(rows_v, out_hbm.at[pl.ds(out_base, CHUNK)])
            return carry

        lax.fori_loop(0, chunks_per_w, body, 0)

    return gather_kernel


def kernel(words, table):
    B = words.shape[0] * words.shape[1]
    D = table.shape[1]
    idx = words.reshape(NUM_WORKERS, B // (NUM_WORKERS * CHUNK), CHUNK)
    out = _make_gather(B, D)(table, idx)
    return out.reshape(words.shape[0], words.shape[1], D)


# speedup vs baseline: 2.7932x; 1.0261x over previous
import functools

import jax
import jax.numpy as jnp
from jax import lax
from jax.experimental import pallas as pl
from jax.experimental.pallas import tpu as pltpu
from jax.experimental.pallas import tpu_sc as plsc

EMBED_DIM = 128
CHUNK = 128
NUM_CORES = 2
NUM_SUBCORES = 16
NUM_WORKERS = NUM_CORES * NUM_SUBCORES


NBUF = 5


@functools.cache
def _make_gather(B: int, D: int):
    assert B % (NUM_WORKERS * CHUNK) == 0
    chunks_per_w = B // (NUM_WORKERS * CHUNK)
    assert chunks_per_w % NBUF == 0
    mesh = plsc.VectorSubcoreMesh(core_axis_name="c", subcore_axis_name="s")

    @functools.partial(
        pl.kernel,
        mesh=mesh,
        out_type=jax.ShapeDtypeStruct((B, D), jnp.float32),
        scratch_types=[
            pltpu.VMEM((chunks_per_w, CHUNK), jnp.int32),
            pltpu.VMEM((NBUF, CHUNK, D), jnp.float32),
            pltpu.SemaphoreType.DMA((NBUF,)),
            pltpu.SemaphoreType.DMA((NBUF,)),
        ],
    )
    def gather_kernel(table_hbm, idx_hbm, out_hbm, idx_v, rows_v, gsem, wsem):
        wid = lax.axis_index("s") * NUM_CORES + lax.axis_index("c")
        pltpu.sync_copy(idx_hbm.at[wid], idx_v)

        @pl.loop(0, chunks_per_w, step=NBUF)
        def superstep(j0):
            gathers = []
            for i in range(NBUF):
                gathers.append(
                    pltpu.async_copy(
                        table_hbm.at[idx_v.at[j0 + i]], rows_v.at[i], gsem.at[i]
                    )
                )
            writes = []
            for i in range(NBUF):
                gathers[i].wait()
                out_base = pl.multiple_of(
                    (wid * chunks_per_w + j0 + i) * CHUNK, CHUNK
                )
                writes.append(
                    pltpu.async_copy(
                        rows_v.at[i], out_hbm.at[pl.ds(out_base, CHUNK)], wsem.at[i]
                    )
                )
            for w in writes:
                w.wait()

    return gather_kernel


def kernel(words, table):
    B = words.shape[0] * words.shape[1]
    D = table.shape[1]
    idx = words.reshape(NUM_WORKERS, B // (NUM_WORKERS * CHUNK), CHUNK)
    out = _make_gather(B, D)(table, idx)
    return out.reshape(words.shape[0], words.shape[1], D)


# speedup vs baseline: 3.5746x; 1.2797x over previous
import functools

import jax
import jax.numpy as jnp
from jax import lax
from jax.experimental import pallas as pl
from jax.experimental.pallas import tpu as pltpu
from jax.experimental.pallas import tpu_sc as plsc

EMBED_DIM = 128
CHUNK = 128
NUM_CORES = 2
NUM_SUBCORES = 16
NUM_WORKERS = NUM_CORES * NUM_SUBCORES


NBUF = 5


@functools.cache
def _make_gather(B: int, D: int, V: int):
    assert B % (NUM_WORKERS * CHUNK) == 0
    chunks_per_w = B // (NUM_WORKERS * CHUNK)
    assert chunks_per_w % NBUF == 0
    mesh = plsc.VectorSubcoreMesh(core_axis_name="c", subcore_axis_name="s")

    @functools.partial(
        pl.kernel,
        mesh=mesh,
        out_type=jax.ShapeDtypeStruct((B, D), jnp.float32),
        scratch_types=[
            pltpu.VMEM((chunks_per_w, CHUNK), jnp.int32),
            pltpu.VMEM((NBUF, CHUNK, D), jnp.float32),
            pltpu.VMEM_SHARED((V, D), jnp.float32),
            pltpu.SemaphoreType.DMA((NBUF,)),
            pltpu.SemaphoreType.DMA((NBUF,)),
        ],
    )
    def gather_kernel(table_hbm, idx_hbm, out_hbm, idx_v, rows_v, table_sp,
                      gsem, wsem):
        wid = lax.axis_index("s") * NUM_CORES + lax.axis_index("c")

        @pl.when(lax.axis_index("s") == 0)
        def _stage_table():
            pltpu.sync_copy(table_hbm, table_sp)

        pltpu.sync_copy(idx_hbm.at[wid], idx_v)
        plsc.subcore_barrier()

        @pl.loop(0, chunks_per_w, step=NBUF)
        def superstep(j0):
            gathers = []
            for i in range(NBUF):
                gathers.append(
                    pltpu.async_copy(
                        table_sp.at[idx_v.at[j0 + i]], rows_v.at[i], gsem.at[i]
                    )
                )
            writes = []
            for i in range(NBUF):
                gathers[i].wait()
                out_base = pl.multiple_of(
                    (wid * chunks_per_w + j0 + i) * CHUNK, CHUNK
                )
                writes.append(
                    pltpu.async_copy(
                        rows_v.at[i], out_hbm.at[pl.ds(out_base, CHUNK)], wsem.at[i]
                    )
                )
            for w in writes:
                w.wait()

    return gather_kernel


def kernel(words, table):
    B = words.shape[0] * words.shape[1]
    D = table.shape[1]
    idx = words.reshape(NUM_WORKERS, B // (NUM_WORKERS * CHUNK), CHUNK)
    out = _make_gather(B, D, table.shape[0])(table, idx)
    return out.reshape(words.shape[0], words.shape[1], D)


# speedup vs baseline: 6.2808x; 1.7571x over previous
import functools

import jax
import jax.numpy as jnp
from jax import lax
from jax.experimental import pallas as pl
from jax.experimental.pallas import tpu as pltpu
from jax.experimental.pallas import tpu_sc as plsc

NUM_CORES = 2
NUM_SUBCORES = 16
NUM_WORKERS = NUM_CORES * NUM_SUBCORES
G = 4
NBUF = 2


@functools.cache
def _make_gather(NB: int, H: int, D: int, V: int):
    assert NB % (NUM_WORKERS * G * NBUF) == 0
    b_per_w = NB // NUM_WORKERS
    n_super = b_per_w // G
    mesh = plsc.VectorSubcoreMesh(core_axis_name="c", subcore_axis_name="s")

    @functools.partial(
        pl.kernel,
        mesh=mesh,
        out_type=jax.ShapeDtypeStruct((NB, H, D), jnp.float32),
        scratch_types=[
            pltpu.VMEM((b_per_w, H), jnp.int32),
            pltpu.VMEM((NBUF, G, H, D), jnp.float32),
            pltpu.VMEM_SHARED((V, D), jnp.float32),
            pltpu.SemaphoreType.DMA((NBUF,)),
            pltpu.SemaphoreType.DMA((NBUF,)),
        ],
    )
    def gather_kernel(table_hbm, idx_hbm, out_hbm, idx_v, rows_v, table_sp,
                      gsem, wsem):
        wid = lax.axis_index("s") * NUM_CORES + lax.axis_index("c")

        @pl.when(lax.axis_index("s") == 0)
        def _stage_table():
            pltpu.sync_copy(table_hbm, table_sp)

        pltpu.sync_copy(idx_hbm.at[wid], idx_v)
        plsc.subcore_barrier()

        @pl.loop(0, n_super, step=NBUF)
        def superstep(s0):
            gathers = []
            for i in range(NBUF):
                e0 = (s0 + i) * G
                for k in range(G):
                    gathers.append(
                        pltpu.async_copy(
                            table_sp.at[idx_v.at[e0 + k]],
                            rows_v.at[i, k],
                            gsem.at[i],
                        )
                    )
            writes = []
            for i in range(NBUF):
                for k in range(G):
                    gathers[i * G + k].wait()
                b_base = pl.multiple_of(wid * b_per_w + (s0 + i) * G, G)
                writes.append(
                    pltpu.async_copy(
                        rows_v.at[i], out_hbm.at[pl.ds(b_base, G)], wsem.at[i]
                    )
                )
            for w in writes:
                w.wait()

    return gather_kernel


def kernel(words, table):
    NB, H = words.shape
    V, D = table.shape
    idx = words.reshape(NUM_WORKERS, NB // NUM_WORKERS, H)
    return _make_gather(NB, H, D, V)(table, idx)
